# baseline (speedup 1.0000x reference)
import functools

import jax
import jax.numpy as jnp
from jax import lax
from jax.experimental import pallas as pl
from jax.experimental.pallas import tpu as pltpu
from jax.experimental.pallas import tpu_sc as plsc

_LAM_U = 0.01
_D = 32
_L = 16
_CHUNK = 128
_MV_W = 65536


def _matvec_body(t_ref, u_ref, o_ref, r_ref):
    i = pl.program_id(0)
    x = t_ref[...]
    u = u_ref[...]
    o_ref[...] = jnp.sum(x * u, axis=0)

    @pl.when(i == 0)
    def _reg():
        r_ref[0, 0] = _LAM_U * jnp.sqrt(jnp.sum(u * u))


@functools.cache
def _matvec_fn(V: int):
    grid = (V + _MV_W - 1) // _MV_W
    return pl.pallas_call(
        _matvec_body,
        grid=(grid,),
        in_specs=[
            pl.BlockSpec((_D, _MV_W), lambda i: (0, i)),
            pl.BlockSpec((_D, 1), lambda i: (0, 0)),
        ],
        out_specs=[
            pl.BlockSpec((_MV_W,), lambda i: (i,)),
            pl.BlockSpec(memory_space=pltpu.SMEM),
        ],
        out_shape=[
            jax.ShapeDtypeStruct((V,), jnp.float32),
            jax.ShapeDtypeStruct((1, 1), jnp.float32),
        ],
    )


def _lane_sum(v):
    lanes = lax.iota(jnp.int32, _L)
    dnums = lax.GatherDimensionNumbers(
        offset_dims=(), collapsed_slice_dims=(0,), start_index_map=(0,))
    for k in (8, 4, 2, 1):
        idx = lax.bitwise_xor(lanes, jnp.full((_L,), k, jnp.int32))
        v = v + lax.gather(v, idx[:, None], dnums, (1,),
                           mode=lax.GatherScatterMode.PROMISE_IN_BOUNDS)
    return v


def _log1p_exp(t):
    e = jnp.exp(t)
    w = e * (1.0 - e * (0.5 - e * (1.0 / 3.0)))
    for _ in range(3):
        w = w - 1.0 + (1.0 + e) * jnp.exp(-w)
    return w


@functools.cache
def _sc_bce_fn(B: int, NC: int, NS: int):
    NW = NC * NS
    b_per_w = B // NW
    n_chunks = b_per_w // _CHUNK
    mesh = plsc.VectorSubcoreMesh(core_axis_name="c", subcore_axis_name="s")

    @functools.partial(
        pl.kernel,
        mesh=mesh,
        compiler_params=pltpu.CompilerParams(use_tc_tiling_on_sc=False),
        out_type=jax.ShapeDtypeStruct((NC, _L), jnp.float32),
        scratch_types=[
            pltpu.VMEM((n_chunks, _CHUNK), jnp.int32),
            pltpu.VMEM((b_per_w,), jnp.float32),
            pltpu.VMEM((b_per_w,), jnp.float32),
            pltpu.VMEM((_L,), jnp.float32),
            pltpu.VMEM((NS, _L), jnp.float32),
            pltpu.VMEM_SHARED((NS, _L), jnp.float32),
            pltpu.SemaphoreType.DMA,
        ],
    )
    def sc_bce(item_hbm, y_hbm, logits_hbm, out_hbm,
               idx_v, g_v, y_v, acc_v, stage_v, shared_v, sem):
        cid = lax.axis_index("c")
        sid = lax.axis_index("s")
        wid = sid * NC + cid
        pltpu.sync_copy(item_hbm.at[wid], idx_v)
        pltpu.sync_copy(y_hbm.at[wid], y_v)
        copies = []
        for j in range(n_chunks):
            copies.append(pltpu.async_copy(
                logits_hbm.at[idx_v.at[j]],
                g_v.at[pl.ds(j * _CHUNK, _CHUNK)],
                sem))
        for c in copies:
            c.wait()

        def body(g, acc):
            x = g_v[pl.ds(g * _L, _L)]
            y = y_v[pl.ds(g * _L, _L)]
            return acc + jnp.maximum(x, 0.0) - x * y + _log1p_exp(-jnp.abs(x))

        acc = lax.fori_loop(0, b_per_w // _L, body,
                            jnp.zeros((_L,), jnp.float32))
        acc_v[...] = acc
        pltpu.sync_copy(acc_v, shared_v.at[sid])
        plsc.subcore_barrier()

        @pl.when(sid == 0)
        def _reduce():
            pltpu.sync_copy(shared_v, stage_v)
            tot = jnp.zeros((_L,), jnp.float32)
            for s in range(NS):
                tot = tot + stage_v[s, :]
            acc_v[...] = _lane_sum(tot)
            pltpu.sync_copy(acc_v, out_hbm.at[cid])

    return sc_bce


def kernel(item, matrix, user_embeddings, item_embeddings):
    B = item.shape[0]
    V = item_embeddings.shape[0]
    try:
        info = plsc.get_sparse_core_info()
        NC, NS = info.num_cores, info.num_subcores
    except Exception:
        NC, NS = 2, 16
    NW = NC * NS
    b_per_w = B // NW
    n_chunks = b_per_w // _CHUNK

    tview = item_embeddings.T
    u_col = user_embeddings.reshape(_D, 1).astype(jnp.float32)
    logits_all, reg = _matvec_fn(V)(tview, u_col)

    item_r = item.astype(jnp.int32).reshape(NW, n_chunks, _CHUNK)
    y_r = matrix.astype(jnp.float32).reshape(NW, b_per_w)
    parts = _sc_bce_fn(B, NC, NS)(item_r, y_r, logits_all)

    return parts[:, 0].sum() + reg[0, 0]
